# initial kernel scaffold (unmeasured)
import jax
import jax.numpy as jnp
from jax import lax
from jax.experimental import pallas as pl
from jax.experimental.pallas import tpu as pltpu

N_DEV = 4
S = 4096
D = 1024
BM = 512
NH = 8
DH = 128
CHUNK = S // N_DEV
EPS = 1e-5
SCALE = 0.08838834764831843
F32 = jnp.float32


def _layernorm_mod(x, s_row, sh_row):
    m = jnp.mean(x, axis=1, keepdims=True)
    v = jnp.mean(x * x, axis=1, keepdims=True) - m * m
    xn = (x - m) * lax.rsqrt(v + EPS)
    return xn * (1.0 + s_row) + sh_row


def _ln_qkv_body(x_ref, s_ref, sh_ref, w_ref, o_ref):
    xm = _layernorm_mod(x_ref[...], s_ref[...], sh_ref[...])
    o_ref[...] = jnp.dot(xm, w_ref[...], preferred_element_type=F32)


def _attn_body(q_ref, k_ref, v_ref, o_ref):
    q = q_ref[...]
    k = k_ref[...]
    s = lax.dot_general(q, k, (((1,), (1,)), ((), ())),
                        preferred_element_type=F32) * SCALE
    m = jnp.max(s, axis=1, keepdims=True)
    p = jnp.exp(s - m)
    l = jnp.sum(p, axis=1, keepdims=True)
    o = jnp.dot(p, v_ref[...], preferred_element_type=F32)
    o_ref[...] = o / l


def _matmul_body(a_ref, b_ref, o_ref):
    o_ref[...] = jnp.dot(a_ref[...], b_ref[...], preferred_element_type=F32)


def _mlp_body(x_ref, s_ref, sh_ref, w1_ref, w2_ref, o_ref):
    xm = _layernorm_mod(x_ref[...], s_ref[...], sh_ref[...])
    h = jnp.dot(xm, w1_ref[...], preferred_element_type=F32)
    h = h * (1.0 / (1.0 + jnp.exp(-h)))
    o_ref[...] = jnp.dot(h, w2_ref[...], preferred_element_type=F32)


def _ar_body(partial_ref, x0_ref, g_ref, o_ref,
             send_buf, recv_buf, send_sems, recv_sems):
    d = lax.axis_index("i")
    left = lax.rem(d + N_DEV - 1, N_DEV)
    right = lax.rem(d + 1, N_DEV)

    barrier = pltpu.get_barrier_semaphore()
    for nbr in (left, right):
        pl.semaphore_signal(barrier, inc=1, device_id=(nbr,),
                            device_id_type=pl.DeviceIdType.MESH)
    pl.semaphore_wait(barrier, 2)

    for s in range(N_DEV - 1):
        c = lax.rem(d - s + N_DEV, N_DEV)
        rows = pl.ds(c * CHUNK, CHUNK)
        if s == 0:
            send_buf[...] = partial_ref[rows, :]
        else:
            send_buf[...] = recv_buf[s - 1] + partial_ref[rows, :]
        rdma = pltpu.make_async_remote_copy(
            src_ref=send_buf,
            dst_ref=recv_buf.at[s],
            send_sem=send_sems.at[s],
            recv_sem=recv_sems.at[s],
            device_id=(right,),
            device_id_type=pl.DeviceIdType.MESH,
        )
        rdma.start()
        rdma.wait()

    own = lax.rem(d + 1, N_DEV)
    own_rows = pl.ds(own * CHUNK, CHUNK)
    o_ref[own_rows, :] = recv_buf[N_DEV - 2] + partial_ref[own_rows, :]

    for t in range(N_DEV - 1):
        c = lax.rem(d + 1 - t + N_DEV, N_DEV)
        rows = pl.ds(c * CHUNK, CHUNK)
        rdma = pltpu.make_async_remote_copy(
            src_ref=o_ref.at[rows, :],
            dst_ref=o_ref.at[rows, :],
            send_sem=send_sems.at[(N_DEV - 1) + t],
            recv_sem=recv_sems.at[(N_DEV - 1) + t],
            device_id=(right,),
            device_id_type=pl.DeviceIdType.MESH,
        )
        rdma.start()
        rdma.wait()

    o_ref[...] = x0_ref[...] + g_ref[...] * o_ref[...]


def _allreduce_residual(partial, x0, gate, cid):
    return pl.pallas_call(
        _ar_body,
        in_specs=[
            pl.BlockSpec(memory_space=pltpu.VMEM),
            pl.BlockSpec(memory_space=pltpu.VMEM),
            pl.BlockSpec(memory_space=pltpu.VMEM),
        ],
        out_specs=pl.BlockSpec(memory_space=pltpu.VMEM),
        out_shape=jax.ShapeDtypeStruct((S, D), F32),
        scratch_shapes=[
            pltpu.VMEM((CHUNK, D), F32),
            pltpu.VMEM((N_DEV - 1, CHUNK, D), F32),
            pltpu.SemaphoreType.DMA((2 * (N_DEV - 1),)),
            pltpu.SemaphoreType.DMA((2 * (N_DEV - 1),)),
        ],
        compiler_params=pltpu.CompilerParams(collective_id=cid),
    )(partial, x0, gate)


def kernel(x, Wq, Wk, Wv, Wo, t_emb, W_mod, W_ff1, W_ff2):
    x2d = x.reshape(S, D)
    mod = jnp.dot(t_emb, W_mod)
    sa, sha, ga, sm, shm, gm = jnp.split(mod, 6, axis=-1)
    Wqkv = jnp.concatenate([Wq, Wk, Wv], axis=1)

    row_specs = [
        pl.BlockSpec((BM, D), lambda i: (i, 0)),
        pl.BlockSpec((1, D), lambda i: (0, 0)),
        pl.BlockSpec((1, D), lambda i: (0, 0)),
    ]

    qkv = pl.pallas_call(
        _ln_qkv_body,
        grid=(S // BM,),
        in_specs=row_specs + [pl.BlockSpec((D, 3 * D), lambda i: (0, 0))],
        out_specs=pl.BlockSpec((BM, 3 * D), lambda i: (i, 0)),
        out_shape=jax.ShapeDtypeStruct((S, 3 * D), F32),
    )(x2d, sa, sha, Wqkv)

    attn = pl.pallas_call(
        _attn_body,
        grid=(NH, S // BM),
        in_specs=[
            pl.BlockSpec((BM, DH), lambda h, q: (q, h)),
            pl.BlockSpec((S, DH), lambda h, q: (0, NH + h)),
            pl.BlockSpec((S, DH), lambda h, q: (0, 2 * NH + h)),
        ],
        out_specs=pl.BlockSpec((BM, DH), lambda h, q: (q, h)),
        out_shape=jax.ShapeDtypeStruct((S, D), F32),
    )(qkv, qkv, qkv)

    attn_partial = pl.pallas_call(
        _matmul_body,
        grid=(S // BM,),
        in_specs=[
            pl.BlockSpec((BM, D), lambda i: (i, 0)),
            pl.BlockSpec((D, D), lambda i: (0, 0)),
        ],
        out_specs=pl.BlockSpec((BM, D), lambda i: (i, 0)),
        out_shape=jax.ShapeDtypeStruct((S, D), F32),
    )(attn, Wo)

    x1 = _allreduce_residual(attn_partial, x2d, ga, cid=0)

    mlp_partial = pl.pallas_call(
        _mlp_body,
        grid=(S // BM,),
        in_specs=row_specs + [
            pl.BlockSpec((D, D), lambda i: (0, 0)),
            pl.BlockSpec((D, D), lambda i: (0, 0)),
        ],
        out_specs=pl.BlockSpec((BM, D), lambda i: (i, 0)),
        out_shape=jax.ShapeDtypeStruct((S, D), F32),
    )(x1, sm, shm, W_ff1, W_ff2)

    out = _allreduce_residual(mlp_partial, x1, gm, cid=1)
    return out.reshape(1, S, D)


# baseline (device time: 1103142 ns/iter reference)
import jax
import jax.numpy as jnp
from jax import lax
from jax.experimental import pallas as pl
from jax.experimental.pallas import tpu as pltpu

N_DEV = 4
S = 4096
D = 1024
BM = 512
NH = 8
DH = 128
CHUNK = S // N_DEV
EPS = 1e-5
SCALE = 0.08838834764831843
F32 = jnp.float32


def _layernorm_mod(x, s_row, sh_row):
    m = jnp.mean(x, axis=1, keepdims=True)
    v = jnp.mean(x * x, axis=1, keepdims=True) - m * m
    xn = (x - m) * lax.rsqrt(v + EPS)
    return xn * (1.0 + s_row) + sh_row


def _ln_qkv_body(x_ref, s_ref, sh_ref, w_ref, o_ref):
    xm = _layernorm_mod(x_ref[...], s_ref[...], sh_ref[...])
    o_ref[...] = jnp.dot(xm, w_ref[...], preferred_element_type=F32)


def _attn_body(q_ref, k_ref, v_ref, o_ref):
    q = q_ref[...]
    k = k_ref[...]
    s = lax.dot_general(q, k, (((1,), (1,)), ((), ())),
                        preferred_element_type=F32) * SCALE
    m = jnp.max(s, axis=1, keepdims=True)
    p = jnp.exp(s - m)
    l = jnp.sum(p, axis=1, keepdims=True)
    o = jnp.dot(p, v_ref[...], preferred_element_type=F32)
    o_ref[...] = o / l


def _matmul_body(a_ref, b_ref, o_ref):
    o_ref[...] = jnp.dot(a_ref[...], b_ref[...], preferred_element_type=F32)


def _mlp_body(x_ref, s_ref, sh_ref, w1_ref, w2_ref, o_ref):
    xm = _layernorm_mod(x_ref[...], s_ref[...], sh_ref[...])
    h = jnp.dot(xm, w1_ref[...], preferred_element_type=F32)
    h = h * (1.0 / (1.0 + jnp.exp(-h)))
    o_ref[...] = jnp.dot(h, w2_ref[...], preferred_element_type=F32)


def _ar_body(partial_ref, x0_ref, g_ref, o_ref,
             recv_buf, send_sems, recv_sems):
    d = lax.axis_index("i")
    left = lax.rem(d + N_DEV - 1, N_DEV)
    right = lax.rem(d + 1, N_DEV)

    barrier = pltpu.get_barrier_semaphore()
    for nbr in (left, right):
        pl.semaphore_signal(barrier, inc=1, device_id=(nbr,),
                            device_id_type=pl.DeviceIdType.MESH)
    pl.semaphore_wait(barrier, 2)

    for s in range(N_DEV - 1):
        c = lax.rem(d - s + N_DEV, N_DEV)
        rows = pl.ds(c * CHUNK, CHUNK)
        if s == 0:
            src = partial_ref.at[rows, :]
        else:
            recv_buf[s - 1] += partial_ref[rows, :]
            src = recv_buf.at[s - 1]
        rdma = pltpu.make_async_remote_copy(
            src_ref=src,
            dst_ref=recv_buf.at[s],
            send_sem=send_sems.at[s],
            recv_sem=recv_sems.at[s],
            device_id=(right,),
            device_id_type=pl.DeviceIdType.MESH,
        )
        rdma.start()
        rdma.wait()

    own = lax.rem(d + 1, N_DEV)
    own_rows = pl.ds(own * CHUNK, CHUNK)
    o_ref[own_rows, :] = recv_buf[N_DEV - 2] + partial_ref[own_rows, :]

    for t in range(N_DEV - 1):
        c = lax.rem(d + 1 - t + N_DEV, N_DEV)
        rows = pl.ds(c * CHUNK, CHUNK)
        rdma = pltpu.make_async_remote_copy(
            src_ref=o_ref.at[rows, :],
            dst_ref=o_ref.at[rows, :],
            send_sem=send_sems.at[(N_DEV - 1) + t],
            recv_sem=recv_sems.at[(N_DEV - 1) + t],
            device_id=(right,),
            device_id_type=pl.DeviceIdType.MESH,
        )
        rdma.start()
        rdma.wait()

    o_ref[...] = x0_ref[...] + g_ref[...] * o_ref[...]


def _allreduce_residual(partial, x0, gate, cid):
    return pl.pallas_call(
        _ar_body,
        in_specs=[
            pl.BlockSpec(memory_space=pltpu.VMEM),
            pl.BlockSpec(memory_space=pltpu.VMEM),
            pl.BlockSpec(memory_space=pltpu.VMEM),
        ],
        out_specs=pl.BlockSpec(memory_space=pltpu.VMEM),
        out_shape=jax.ShapeDtypeStruct((S, D), F32),
        scratch_shapes=[
            pltpu.VMEM((N_DEV - 1, CHUNK, D), F32),
            pltpu.SemaphoreType.DMA((2 * (N_DEV - 1),)),
            pltpu.SemaphoreType.DMA((2 * (N_DEV - 1),)),
        ],
        compiler_params=pltpu.CompilerParams(
            collective_id=cid, vmem_limit_bytes=112 * 1024 * 1024
        ),
    )(partial, x0, gate)


def kernel(x, Wq, Wk, Wv, Wo, t_emb, W_mod, W_ff1, W_ff2):
    x2d = x.reshape(S, D)
    mod = jnp.dot(t_emb, W_mod)
    sa, sha, ga, sm, shm, gm = jnp.split(mod, 6, axis=-1)
    Wqkv = jnp.concatenate([Wq, Wk, Wv], axis=1)

    row_specs = [
        pl.BlockSpec((BM, D), lambda i: (i, 0)),
        pl.BlockSpec((1, D), lambda i: (0, 0)),
        pl.BlockSpec((1, D), lambda i: (0, 0)),
    ]

    qkv = pl.pallas_call(
        _ln_qkv_body,
        grid=(S // BM,),
        in_specs=row_specs + [pl.BlockSpec((D, 3 * D), lambda i: (0, 0))],
        out_specs=pl.BlockSpec((BM, 3 * D), lambda i: (i, 0)),
        out_shape=jax.ShapeDtypeStruct((S, 3 * D), F32),
    )(x2d, sa, sha, Wqkv)

    attn = pl.pallas_call(
        _attn_body,
        grid=(NH, S // BM),
        in_specs=[
            pl.BlockSpec((BM, DH), lambda h, q: (q, h)),
            pl.BlockSpec((S, DH), lambda h, q: (0, NH + h)),
            pl.BlockSpec((S, DH), lambda h, q: (0, 2 * NH + h)),
        ],
        out_specs=pl.BlockSpec((BM, DH), lambda h, q: (q, h)),
        out_shape=jax.ShapeDtypeStruct((S, D), F32),
    )(qkv, qkv, qkv)

    attn_partial = pl.pallas_call(
        _matmul_body,
        grid=(S // BM,),
        in_specs=[
            pl.BlockSpec((BM, D), lambda i: (i, 0)),
            pl.BlockSpec((D, D), lambda i: (0, 0)),
        ],
        out_specs=pl.BlockSpec((BM, D), lambda i: (i, 0)),
        out_shape=jax.ShapeDtypeStruct((S, D), F32),
    )(attn, Wo)

    x1 = _allreduce_residual(attn_partial, x2d, ga, cid=0)

    mlp_partial = pl.pallas_call(
        _mlp_body,
        grid=(S // BM,),
        in_specs=row_specs + [
            pl.BlockSpec((D, D), lambda i: (0, 0)),
            pl.BlockSpec((D, D), lambda i: (0, 0)),
        ],
        out_specs=pl.BlockSpec((BM, D), lambda i: (i, 0)),
        out_shape=jax.ShapeDtypeStruct((S, D), F32),
    )(x1, sm, shm, W_ff1, W_ff2)

    out = _allreduce_residual(mlp_partial, x1, gm, cid=1)
    return out.reshape(1, S, D)


# device time: 837631 ns/iter; 1.3170x vs baseline; 1.3170x over previous
import jax
import jax.numpy as jnp
from jax import lax
from jax.experimental import pallas as pl
from jax.experimental.pallas import tpu as pltpu

N_DEV = 4
S = 4096
D = 1024
BM = 512
NH = 8
DH = 128
CHUNK = S // N_DEV
EPS = 1e-5
SCALE = 0.08838834764831843
F32 = jnp.float32


def _layernorm_mod(x, s_row, sh_row):
    m = jnp.mean(x, axis=1, keepdims=True)
    v = jnp.mean(x * x, axis=1, keepdims=True) - m * m
    xn = (x - m) * lax.rsqrt(v + EPS)
    return xn * (1.0 + s_row) + sh_row


def _ln_qkv_body(x_ref, s_ref, sh_ref, w_ref, o_ref):
    xm = _layernorm_mod(x_ref[...], s_ref[...], sh_ref[...])
    o_ref[...] = jnp.dot(xm, w_ref[...], preferred_element_type=F32)


def _attn_body(q_ref, k_ref, v_ref, o_ref):
    q = q_ref[...].astype(jnp.bfloat16)
    k = k_ref[...].astype(jnp.bfloat16)
    s = lax.dot_general(q, k, (((1,), (1,)), ((), ())),
                        preferred_element_type=F32) * SCALE
    m = jnp.max(s, axis=1, keepdims=True)
    p = jnp.exp(s - m)
    l = jnp.sum(p, axis=1, keepdims=True)
    o = jnp.dot(p.astype(jnp.bfloat16), v_ref[...].astype(jnp.bfloat16),
                preferred_element_type=F32)
    o_ref[...] = o / l


def _matmul_body(a_ref, b_ref, o_ref):
    o_ref[...] = jnp.dot(a_ref[...], b_ref[...], preferred_element_type=F32)


def _mlp_body(x_ref, s_ref, sh_ref, w1_ref, w2_ref, o_ref):
    xm = _layernorm_mod(x_ref[...], s_ref[...], sh_ref[...])
    h = jnp.dot(xm, w1_ref[...], preferred_element_type=F32)
    h = h * (1.0 / (1.0 + jnp.exp(-h)))
    o_ref[...] = jnp.dot(h, w2_ref[...], preferred_element_type=F32)


HCHUNK = S // (2 * N_DEV)


def _ar_body(partial_ref, x0_ref, g_ref, o_ref,
             recv_buf, send_sems, recv_sems):
    d = lax.axis_index("i")
    left = lax.rem(d + N_DEV - 1, N_DEV)
    right = lax.rem(d + 1, N_DEV)

    barrier = pltpu.get_barrier_semaphore()
    for nbr in (left, right):
        pl.semaphore_signal(barrier, inc=1, device_id=(nbr,),
                            device_id_type=pl.DeviceIdType.MESH)
    pl.semaphore_wait(barrier, 2)

    def rows_cw(c):
        return pl.ds(c * HCHUNK, HCHUNK)

    def rows_ccw(c):
        return pl.ds(S // 2 + c * HCHUNK, HCHUNK)

    for s in range(N_DEV - 1):
        rdmas = []
        for dir_, rows_of, nbr in ((0, rows_cw, right), (1, rows_ccw, left)):
            if dir_ == 0:
                c = lax.rem(d - s + N_DEV, N_DEV)
            else:
                c = lax.rem(d + s, N_DEV)
            rows = rows_of(c)
            if s == 0:
                src = partial_ref.at[rows, :]
            else:
                recv_buf[dir_, s - 1] += partial_ref[rows, :]
                src = recv_buf.at[dir_, s - 1]
            rdma = pltpu.make_async_remote_copy(
                src_ref=src,
                dst_ref=recv_buf.at[dir_, s],
                send_sem=send_sems.at[dir_, s],
                recv_sem=recv_sems.at[dir_, s],
                device_id=(nbr,),
                device_id_type=pl.DeviceIdType.MESH,
            )
            rdma.start()
            rdmas.append(rdma)
        for rdma in rdmas:
            rdma.wait()

    own_cw = lax.rem(d + 1, N_DEV)
    o_ref[rows_cw(own_cw), :] = (
        recv_buf[0, N_DEV - 2] + partial_ref[rows_cw(own_cw), :]
    )
    own_ccw = lax.rem(d + N_DEV - 1, N_DEV)
    o_ref[rows_ccw(own_ccw), :] = (
        recv_buf[1, N_DEV - 2] + partial_ref[rows_ccw(own_ccw), :]
    )

    for t in range(N_DEV - 1):
        rdmas = []
        for dir_, rows_of, nbr in ((0, rows_cw, right), (1, rows_ccw, left)):
            if dir_ == 0:
                c = lax.rem(d + 1 - t + N_DEV, N_DEV)
            else:
                c = lax.rem(d + N_DEV - 1 + t, N_DEV)
            rows = rows_of(c)
            rdma = pltpu.make_async_remote_copy(
                src_ref=o_ref.at[rows, :],
                dst_ref=o_ref.at[rows, :],
                send_sem=send_sems.at[dir_, (N_DEV - 1) + t],
                recv_sem=recv_sems.at[dir_, (N_DEV - 1) + t],
                device_id=(nbr,),
                device_id_type=pl.DeviceIdType.MESH,
            )
            rdma.start()
            rdmas.append(rdma)
        for rdma in rdmas:
            rdma.wait()

    o_ref[...] = x0_ref[...] + g_ref[...] * o_ref[...]


def _allreduce_residual(partial, x0, gate, cid):
    return pl.pallas_call(
        _ar_body,
        in_specs=[
            pl.BlockSpec(memory_space=pltpu.VMEM),
            pl.BlockSpec(memory_space=pltpu.VMEM),
            pl.BlockSpec(memory_space=pltpu.VMEM),
        ],
        out_specs=pl.BlockSpec(memory_space=pltpu.VMEM),
        out_shape=jax.ShapeDtypeStruct((S, D), F32),
        scratch_shapes=[
            pltpu.VMEM((2, N_DEV - 1, HCHUNK, D), F32),
            pltpu.SemaphoreType.DMA((2, 2 * (N_DEV - 1))),
            pltpu.SemaphoreType.DMA((2, 2 * (N_DEV - 1))),
        ],
        compiler_params=pltpu.CompilerParams(
            collective_id=cid, vmem_limit_bytes=112 * 1024 * 1024
        ),
    )(partial, x0, gate)


def kernel(x, Wq, Wk, Wv, Wo, t_emb, W_mod, W_ff1, W_ff2):
    x2d = x.reshape(S, D)
    mod = jnp.dot(t_emb, W_mod)
    sa, sha, ga, sm, shm, gm = jnp.split(mod, 6, axis=-1)
    Wqkv = jnp.concatenate([Wq, Wk, Wv], axis=1)

    row_specs = [
        pl.BlockSpec((BM, D), lambda i: (i, 0)),
        pl.BlockSpec((1, D), lambda i: (0, 0)),
        pl.BlockSpec((1, D), lambda i: (0, 0)),
    ]

    qkv = pl.pallas_call(
        _ln_qkv_body,
        grid=(S // BM,),
        in_specs=row_specs + [pl.BlockSpec((D, 3 * D), lambda i: (0, 0))],
        out_specs=pl.BlockSpec((BM, 3 * D), lambda i: (i, 0)),
        out_shape=jax.ShapeDtypeStruct((S, 3 * D), F32),
    )(x2d, sa, sha, Wqkv)

    attn = pl.pallas_call(
        _attn_body,
        grid=(NH, S // BM),
        in_specs=[
            pl.BlockSpec((BM, DH), lambda h, q: (q, h)),
            pl.BlockSpec((S, DH), lambda h, q: (0, NH + h)),
            pl.BlockSpec((S, DH), lambda h, q: (0, 2 * NH + h)),
        ],
        out_specs=pl.BlockSpec((BM, DH), lambda h, q: (q, h)),
        out_shape=jax.ShapeDtypeStruct((S, D), F32),
    )(qkv, qkv, qkv)

    attn_partial = pl.pallas_call(
        _matmul_body,
        grid=(S // BM,),
        in_specs=[
            pl.BlockSpec((BM, D), lambda i: (i, 0)),
            pl.BlockSpec((D, D), lambda i: (0, 0)),
        ],
        out_specs=pl.BlockSpec((BM, D), lambda i: (i, 0)),
        out_shape=jax.ShapeDtypeStruct((S, D), F32),
    )(attn, Wo)

    x1 = _allreduce_residual(attn_partial, x2d, ga, cid=0)

    mlp_partial = pl.pallas_call(
        _mlp_body,
        grid=(S // BM,),
        in_specs=row_specs + [
            pl.BlockSpec((D, D), lambda i: (0, 0)),
            pl.BlockSpec((D, D), lambda i: (0, 0)),
        ],
        out_specs=pl.BlockSpec((BM, D), lambda i: (i, 0)),
        out_shape=jax.ShapeDtypeStruct((S, D), F32),
    )(x1, sm, shm, W_ff1, W_ff2)

    out = _allreduce_residual(mlp_partial, x1, gm, cid=1)
    return out.reshape(1, S, D)


# device time: 481101 ns/iter; 2.2930x vs baseline; 1.7411x over previous
import jax
import jax.numpy as jnp
from jax import lax
from jax.experimental import pallas as pl
from jax.experimental.pallas import tpu as pltpu

N_DEV = 4
S = 4096
D = 1024
BM = 512
NH = 8
DH = 128
CHUNK = S // N_DEV
EPS = 1e-5
SCALE = 0.08838834764831843
F32 = jnp.float32


def _layernorm_mod(x, s_row, sh_row):
    m = jnp.mean(x, axis=1, keepdims=True)
    v = jnp.mean(x * x, axis=1, keepdims=True) - m * m
    xn = (x - m) * lax.rsqrt(v + EPS)
    return xn * (1.0 + s_row) + sh_row


def _ln_qkv_body(x_ref, s_ref, sh_ref, w_ref, o_ref):
    xm = _layernorm_mod(x_ref[...], s_ref[...], sh_ref[...])
    o_ref[...] = jnp.dot(xm, w_ref[...], preferred_element_type=F32)


def _attn_body(q_ref, k_ref, v_ref, o_ref):
    q = (q_ref[...] * SCALE).astype(jnp.bfloat16)
    k = k_ref[...].astype(jnp.bfloat16)
    s = lax.dot_general(q, k, (((1,), (1,)), ((), ())),
                        preferred_element_type=F32)
    p = jnp.exp(s)
    l = jnp.sum(p, axis=1, keepdims=True)
    o = jnp.dot(p.astype(jnp.bfloat16), v_ref[...].astype(jnp.bfloat16),
                preferred_element_type=F32)
    o_ref[...] = o / l


def _matmul_body(a_ref, b_ref, o_ref):
    o_ref[...] = jnp.dot(a_ref[...], b_ref[...], preferred_element_type=F32)


def _mlp_body(x_ref, s_ref, sh_ref, w1_ref, w2_ref, o_ref):
    xm = _layernorm_mod(x_ref[...], s_ref[...], sh_ref[...])
    h = jnp.dot(xm, w1_ref[...], preferred_element_type=F32)
    h = h * (1.0 / (1.0 + jnp.exp(-h)))
    o_ref[...] = jnp.dot(h, w2_ref[...], preferred_element_type=F32)


HCHUNK = S // (2 * N_DEV)


def _ar_body(partial_ref, x0_ref, g_ref, o_ref,
             recv_buf, stage_buf, ag_buf, send_sems, recv_sems):
    d = lax.axis_index("i")
    left = lax.rem(d + N_DEV - 1, N_DEV)
    right = lax.rem(d + 1, N_DEV)

    barrier = pltpu.get_barrier_semaphore()
    for nbr in (left, right):
        pl.semaphore_signal(barrier, inc=1, device_id=(nbr,),
                            device_id_type=pl.DeviceIdType.MESH)
    pl.semaphore_wait(barrier, 2)

    def rows_cw(c):
        return pl.ds(c * HCHUNK, HCHUNK)

    def rows_ccw(c):
        return pl.ds(S // 2 + c * HCHUNK, HCHUNK)

    for s in range(N_DEV - 1):
        rdmas = []
        for dir_, rows_of, nbr in ((0, rows_cw, right), (1, rows_ccw, left)):
            if dir_ == 0:
                c = lax.rem(d - s + N_DEV, N_DEV)
            else:
                c = lax.rem(d + s, N_DEV)
            rows = rows_of(c)
            if s == 0:
                acc = partial_ref[rows, :]
            else:
                acc = recv_buf[dir_, s - 1].astype(F32) + partial_ref[rows, :]
            stage_buf[dir_] = acc.astype(jnp.bfloat16)
            rdma = pltpu.make_async_remote_copy(
                src_ref=stage_buf.at[dir_],
                dst_ref=recv_buf.at[dir_, s],
                send_sem=send_sems.at[dir_, s],
                recv_sem=recv_sems.at[dir_, s],
                device_id=(nbr,),
                device_id_type=pl.DeviceIdType.MESH,
            )
            rdma.start()
            rdmas.append(rdma)
        for rdma in rdmas:
            rdma.wait()

    own_cw = lax.rem(d + 1, N_DEV)
    red_cw = recv_buf[0, N_DEV - 2].astype(F32) + partial_ref[rows_cw(own_cw), :]
    o_ref[rows_cw(own_cw), :] = red_cw
    stage_buf[0] = red_cw.astype(jnp.bfloat16)
    own_ccw = lax.rem(d + N_DEV - 1, N_DEV)
    red_ccw = recv_buf[1, N_DEV - 2].astype(F32) + partial_ref[rows_ccw(own_ccw), :]
    o_ref[rows_ccw(own_ccw), :] = red_ccw
    stage_buf[1] = red_ccw.astype(jnp.bfloat16)

    for t in range(N_DEV - 1):
        rdmas = []
        for dir_, nbr in ((0, right), (1, left)):
            src = stage_buf.at[dir_] if t == 0 else ag_buf.at[dir_, t - 1]
            rdma = pltpu.make_async_remote_copy(
                src_ref=src,
                dst_ref=ag_buf.at[dir_, t],
                send_sem=send_sems.at[dir_, (N_DEV - 1) + t],
                recv_sem=recv_sems.at[dir_, (N_DEV - 1) + t],
                device_id=(nbr,),
                device_id_type=pl.DeviceIdType.MESH,
            )
            rdma.start()
            rdmas.append(rdma)
        for rdma in rdmas:
            rdma.wait()
        c_cw = lax.rem(d - t + N_DEV, N_DEV)
        o_ref[rows_cw(c_cw), :] = ag_buf[0, t].astype(F32)
        c_ccw = lax.rem(d + t, N_DEV)
        o_ref[rows_ccw(c_ccw), :] = ag_buf[1, t].astype(F32)

    o_ref[...] = x0_ref[...] + g_ref[...] * o_ref[...]


def _allreduce_residual(partial, x0, gate, cid):
    return pl.pallas_call(
        _ar_body,
        in_specs=[
            pl.BlockSpec(memory_space=pltpu.VMEM),
            pl.BlockSpec(memory_space=pltpu.VMEM),
            pl.BlockSpec(memory_space=pltpu.VMEM),
        ],
        out_specs=pl.BlockSpec(memory_space=pltpu.VMEM),
        out_shape=jax.ShapeDtypeStruct((S, D), F32),
        scratch_shapes=[
            pltpu.VMEM((2, N_DEV - 1, HCHUNK, D), jnp.bfloat16),
            pltpu.VMEM((2, HCHUNK, D), jnp.bfloat16),
            pltpu.VMEM((2, N_DEV - 1, HCHUNK, D), jnp.bfloat16),
            pltpu.SemaphoreType.DMA((2, 2 * (N_DEV - 1))),
            pltpu.SemaphoreType.DMA((2, 2 * (N_DEV - 1))),
        ],
        compiler_params=pltpu.CompilerParams(
            collective_id=cid, vmem_limit_bytes=112 * 1024 * 1024
        ),
    )(partial, x0, gate)


def kernel(x, Wq, Wk, Wv, Wo, t_emb, W_mod, W_ff1, W_ff2):
    x2d = x.reshape(S, D)
    mod = jnp.dot(t_emb, W_mod)
    sa, sha, ga, sm, shm, gm = jnp.split(mod, 6, axis=-1)
    Wqkv = jnp.concatenate([Wq, Wk, Wv], axis=1)

    row_specs = [
        pl.BlockSpec((BM, D), lambda i: (i, 0)),
        pl.BlockSpec((1, D), lambda i: (0, 0)),
        pl.BlockSpec((1, D), lambda i: (0, 0)),
    ]

    qkv = pl.pallas_call(
        _ln_qkv_body,
        grid=(S // BM,),
        in_specs=row_specs + [pl.BlockSpec((D, 3 * D), lambda i: (0, 0))],
        out_specs=pl.BlockSpec((BM, 3 * D), lambda i: (i, 0)),
        out_shape=jax.ShapeDtypeStruct((S, 3 * D), F32),
    )(x2d, sa, sha, Wqkv)

    attn = pl.pallas_call(
        _attn_body,
        grid=(NH, S // BM),
        in_specs=[
            pl.BlockSpec((BM, DH), lambda h, q: (q, h)),
            pl.BlockSpec((S, DH), lambda h, q: (0, NH + h)),
            pl.BlockSpec((S, DH), lambda h, q: (0, 2 * NH + h)),
        ],
        out_specs=pl.BlockSpec((BM, DH), lambda h, q: (q, h)),
        out_shape=jax.ShapeDtypeStruct((S, D), F32),
    )(qkv, qkv, qkv)

    attn_partial = pl.pallas_call(
        _matmul_body,
        grid=(S // BM,),
        in_specs=[
            pl.BlockSpec((BM, D), lambda i: (i, 0)),
            pl.BlockSpec((D, D), lambda i: (0, 0)),
        ],
        out_specs=pl.BlockSpec((BM, D), lambda i: (i, 0)),
        out_shape=jax.ShapeDtypeStruct((S, D), F32),
    )(attn, Wo)

    x1 = _allreduce_residual(attn_partial, x2d, ga, cid=0)

    mlp_partial = pl.pallas_call(
        _mlp_body,
        grid=(S // BM,),
        in_specs=row_specs + [
            pl.BlockSpec((D, D), lambda i: (0, 0)),
            pl.BlockSpec((D, D), lambda i: (0, 0)),
        ],
        out_specs=pl.BlockSpec((BM, D), lambda i: (i, 0)),
        out_shape=jax.ShapeDtypeStruct((S, D), F32),
    )(x1, sm, shm, W_ff1, W_ff2)

    out = _allreduce_residual(mlp_partial, x1, gm, cid=1)
    return out.reshape(1, S, D)


# device time: 471759 ns/iter; 2.3384x vs baseline; 1.0198x over previous
import jax
import jax.numpy as jnp
from jax import lax
from jax.experimental import pallas as pl
from jax.experimental.pallas import tpu as pltpu

N_DEV = 4
S = 4096
D = 1024
BM = 512
NH = 8
DH = 128
CHUNK = S // N_DEV
EPS = 1e-5
SCALE = 0.08838834764831843
F32 = jnp.float32


def _layernorm_mod(x, s_row, sh_row):
    m = jnp.mean(x, axis=1, keepdims=True)
    v = jnp.mean(x * x, axis=1, keepdims=True) - m * m
    xn = (x - m) * lax.rsqrt(v + EPS)
    return xn * (1.0 + s_row) + sh_row


def _ln_qkv_body(x_ref, s_ref, sh_ref, wq_ref, wk_ref, wv_ref, o_ref):
    xm = _layernorm_mod(x_ref[...], s_ref[...], sh_ref[...])
    xb = xm.astype(jnp.bfloat16)
    for j, w_ref in enumerate((wq_ref, wk_ref, wv_ref)):
        o_ref[:, pl.ds(j * D, D)] = jnp.dot(
            xb, w_ref[...].astype(jnp.bfloat16),
            preferred_element_type=F32,
        ).astype(jnp.bfloat16)


def _attn_body(q_ref, k_ref, v_ref, o_ref):
    q = q_ref[...] * jnp.bfloat16(SCALE * 1.4426950408889634)
    s = lax.dot_general(q, k_ref[...], (((1,), (1,)), ((), ())),
                        preferred_element_type=F32)
    p = jnp.exp2(s)
    l = jnp.sum(p, axis=1, keepdims=True)
    o = jnp.dot(p.astype(jnp.bfloat16), v_ref[...],
                preferred_element_type=F32)
    o_ref[...] = (o / l).astype(jnp.bfloat16)


def _matmul_body(a_ref, b_ref, o_ref):
    o_ref[...] = jnp.dot(a_ref[...], b_ref[...].astype(jnp.bfloat16),
                         preferred_element_type=F32)


def _mlp_body(x_ref, s_ref, sh_ref, w1_ref, w2_ref, o_ref):
    xm = _layernorm_mod(x_ref[...], s_ref[...], sh_ref[...])
    h = jnp.dot(xm.astype(jnp.bfloat16), w1_ref[...].astype(jnp.bfloat16),
                preferred_element_type=F32)
    h = h * (1.0 / (1.0 + jnp.exp(-h)))
    o_ref[...] = jnp.dot(h.astype(jnp.bfloat16),
                         w2_ref[...].astype(jnp.bfloat16),
                         preferred_element_type=F32)


HCHUNK = S // (2 * N_DEV)


def _ar_body(partial_ref, x0_ref, g_ref, o_ref,
             recv_buf, stage_buf, ag_buf, send_sems, recv_sems):
    d = lax.axis_index("i")
    left = lax.rem(d + N_DEV - 1, N_DEV)
    right = lax.rem(d + 1, N_DEV)

    barrier = pltpu.get_barrier_semaphore()
    for nbr in (left, right):
        pl.semaphore_signal(barrier, inc=1, device_id=(nbr,),
                            device_id_type=pl.DeviceIdType.MESH)
    pl.semaphore_wait(barrier, 2)

    def rows_cw(c):
        return pl.ds(c * HCHUNK, HCHUNK)

    def rows_ccw(c):
        return pl.ds(S // 2 + c * HCHUNK, HCHUNK)

    for s in range(N_DEV - 1):
        rdmas = []
        for dir_, rows_of, nbr in ((0, rows_cw, right), (1, rows_ccw, left)):
            if dir_ == 0:
                c = lax.rem(d - s + N_DEV, N_DEV)
            else:
                c = lax.rem(d + s, N_DEV)
            rows = rows_of(c)
            if s == 0:
                acc = partial_ref[rows, :]
            else:
                acc = recv_buf[dir_, s - 1].astype(F32) + partial_ref[rows, :]
            stage_buf[dir_] = acc.astype(jnp.bfloat16)
            rdma = pltpu.make_async_remote_copy(
                src_ref=stage_buf.at[dir_],
                dst_ref=recv_buf.at[dir_, s],
                send_sem=send_sems.at[dir_, s],
                recv_sem=recv_sems.at[dir_, s],
                device_id=(nbr,),
                device_id_type=pl.DeviceIdType.MESH,
            )
            rdma.start()
            rdmas.append(rdma)
        for rdma in rdmas:
            rdma.wait()

    own_cw = lax.rem(d + 1, N_DEV)
    red_cw = recv_buf[0, N_DEV - 2].astype(F32) + partial_ref[rows_cw(own_cw), :]
    o_ref[rows_cw(own_cw), :] = red_cw
    stage_buf[0] = red_cw.astype(jnp.bfloat16)
    own_ccw = lax.rem(d + N_DEV - 1, N_DEV)
    red_ccw = recv_buf[1, N_DEV - 2].astype(F32) + partial_ref[rows_ccw(own_ccw), :]
    o_ref[rows_ccw(own_ccw), :] = red_ccw
    stage_buf[1] = red_ccw.astype(jnp.bfloat16)

    for t in range(N_DEV - 1):
        rdmas = []
        for dir_, nbr in ((0, right), (1, left)):
            src = stage_buf.at[dir_] if t == 0 else ag_buf.at[dir_, t - 1]
            rdma = pltpu.make_async_remote_copy(
                src_ref=src,
                dst_ref=ag_buf.at[dir_, t],
                send_sem=send_sems.at[dir_, (N_DEV - 1) + t],
                recv_sem=recv_sems.at[dir_, (N_DEV - 1) + t],
                device_id=(nbr,),
                device_id_type=pl.DeviceIdType.MESH,
            )
            rdma.start()
            rdmas.append(rdma)
        for rdma in rdmas:
            rdma.wait()
        c_cw = lax.rem(d - t + N_DEV, N_DEV)
        o_ref[rows_cw(c_cw), :] = ag_buf[0, t].astype(F32)
        c_ccw = lax.rem(d + t, N_DEV)
        o_ref[rows_ccw(c_ccw), :] = ag_buf[1, t].astype(F32)

    o_ref[...] = x0_ref[...] + g_ref[...] * o_ref[...]


def _allreduce_residual(partial, x0, gate, cid):
    return pl.pallas_call(
        _ar_body,
        in_specs=[
            pl.BlockSpec(memory_space=pltpu.VMEM),
            pl.BlockSpec(memory_space=pltpu.VMEM),
            pl.BlockSpec(memory_space=pltpu.VMEM),
        ],
        out_specs=pl.BlockSpec(memory_space=pltpu.VMEM),
        out_shape=jax.ShapeDtypeStruct((S, D), F32),
        scratch_shapes=[
            pltpu.VMEM((2, N_DEV - 1, HCHUNK, D), jnp.bfloat16),
            pltpu.VMEM((2, HCHUNK, D), jnp.bfloat16),
            pltpu.VMEM((2, N_DEV - 1, HCHUNK, D), jnp.bfloat16),
            pltpu.SemaphoreType.DMA((2, 2 * (N_DEV - 1))),
            pltpu.SemaphoreType.DMA((2, 2 * (N_DEV - 1))),
        ],
        compiler_params=pltpu.CompilerParams(
            collective_id=cid, vmem_limit_bytes=112 * 1024 * 1024
        ),
    )(partial, x0, gate)


def kernel(x, Wq, Wk, Wv, Wo, t_emb, W_mod, W_ff1, W_ff2):
    x2d = x.reshape(S, D)
    mod = jnp.dot(t_emb, W_mod)
    sa, sha, ga, sm, shm, gm = jnp.split(mod, 6, axis=-1)

    row_specs = [
        pl.BlockSpec((BM, D), lambda i: (i, 0)),
        pl.BlockSpec((1, D), lambda i: (0, 0)),
        pl.BlockSpec((1, D), lambda i: (0, 0)),
    ]

    qkv = pl.pallas_call(
        _ln_qkv_body,
        grid=(S // BM,),
        in_specs=row_specs + [pl.BlockSpec((D, D), lambda i: (0, 0))] * 3,
        out_specs=pl.BlockSpec((BM, 3 * D), lambda i: (i, 0)),
        out_shape=jax.ShapeDtypeStruct((S, 3 * D), jnp.bfloat16),
    )(x2d, sa, sha, Wq, Wk, Wv)

    attn = pl.pallas_call(
        _attn_body,
        grid=(NH, S // BM),
        in_specs=[
            pl.BlockSpec((BM, DH), lambda h, q: (q, h)),
            pl.BlockSpec((S, DH), lambda h, q: (0, NH + h)),
            pl.BlockSpec((S, DH), lambda h, q: (0, 2 * NH + h)),
        ],
        out_specs=pl.BlockSpec((BM, DH), lambda h, q: (q, h)),
        out_shape=jax.ShapeDtypeStruct((S, D), jnp.bfloat16),
    )(qkv, qkv, qkv)

    attn_partial = pl.pallas_call(
        _matmul_body,
        grid=(S // BM,),
        in_specs=[
            pl.BlockSpec((BM, D), lambda i: (i, 0)),
            pl.BlockSpec((D, D), lambda i: (0, 0)),
        ],
        out_specs=pl.BlockSpec((BM, D), lambda i: (i, 0)),
        out_shape=jax.ShapeDtypeStruct((S, D), F32),
    )(attn, Wo)

    x1 = _allreduce_residual(attn_partial, x2d, ga, cid=0)

    mlp_partial = pl.pallas_call(
        _mlp_body,
        grid=(S // BM,),
        in_specs=row_specs + [
            pl.BlockSpec((D, D), lambda i: (0, 0)),
            pl.BlockSpec((D, D), lambda i: (0, 0)),
        ],
        out_specs=pl.BlockSpec((BM, D), lambda i: (i, 0)),
        out_shape=jax.ShapeDtypeStruct((S, D), F32),
    )(x1, sm, shm, W_ff1, W_ff2)

    out = _allreduce_residual(mlp_partial, x1, gm, cid=1)
    return out.reshape(1, S, D)
